# Initial kernel scaffold; baseline (speedup 1.0000x reference)
import jax
import jax.numpy as jnp
from jax.experimental import pallas as pl


def kernel(x, lift_W, lift_b, proj_W, proj_b, sp_emb, sp_W, sp_b, fr_emb, fr_W, fr_b, g_W1, g_b1, g_W2, g_b2, mhf_Wr, mhf_Wi):
    raise NotImplementedError("write your pallas kernel here")


# speedup vs baseline: 8.7751x; 8.7751x over previous
import numpy as np
import jax
import jax.numpy as jnp
from jax.experimental import pallas as pl
from jax.experimental.pallas import tpu as pltpu

L = 4
HID = 64
MODES = 16
HEADS = 4
CH = HID // HEADS
NP_SP = 500
NP_FR = 200
ED = 8
PRIME = 31.0
S = 16384
B = 16
NBINS = 512

_HI = jax.lax.Precision.HIGHEST
_H3 = jax.lax.Precision.HIGH

_s = np.arange(S, dtype=np.int64)
_k = np.arange(MODES, dtype=np.int64)
_ang = 2.0 * np.pi * ((_s[:, None] * _k[None, :]) % S).astype(np.float64) / S
_FC = np.cos(_ang).astype(np.float32)
_FS = (-np.sin(_ang)).astype(np.float32)
_wk = np.full((MODES, 1), 2.0 / S)
_wk[0, 0] = 1.0 / S
_IC = (np.cos(_ang).T * _wk).astype(np.float32)
_IS = (-np.sin(_ang).T * _wk).astype(np.float32)


def _dot(a, b, prec=_HI):
    return jax.lax.dot(a, b, precision=prec, preferred_element_type=jnp.float32)


_DEF = jax.lax.Precision.DEFAULT


def _r(x):
    return x.astype(jnp.bfloat16).astype(jnp.float32)


def _analysis(h, fc_ref, fs_ref, fr_ref, fi_ref, idx_ref, hist_ref, hist_scr):
    fr_ref[0] = _dot(h, fc_ref[...])
    fi_ref[0] = _dot(h, fs_ref[...])
    cs = jnp.sum(h, axis=0, keepdims=True)
    c0 = cs[0, 0]
    cL = cs[0, S - 1]
    iot = jax.lax.broadcasted_iota(jnp.int32, (1, S), 1)
    r2 = jnp.roll(cs, 2, axis=1)
    r1 = jnp.roll(cs, 1, axis=1)
    rm1 = jnp.roll(cs, -1, axis=1)
    a = jnp.where(iot >= 2, r2, c0)
    bt = jnp.where(iot >= 1, r1, c0)
    d = jnp.where(iot <= S - 2, rm1, cL)
    wsum = a + bt + cs + d
    idx = (wsum * PRIME).astype(jnp.int32) % NP_SP
    idx = jnp.clip(idx, 0, NP_SP - 1)
    idx_ref[0] = idx
    hist_scr[...] = jnp.zeros((NBINS, 128), jnp.float32)
    bins = jax.lax.broadcasted_iota(jnp.int32, (NBINS, 1), 0)

    def hbody(j, _):
        ch = idx_ref[0, :, pl.ds(j * 128, 128)]
        hist_scr[...] += (bins == ch).astype(jnp.float32)
        return 0

    jax.lax.fori_loop(0, S // 128, hbody, 0)
    ht = jnp.transpose(hist_scr[...])
    hist_ref[0] = jnp.sum(ht, axis=0, keepdims=True)


def _apply(idx_in_ref, mr_ref, mi_ref, wg_ref, tt_ref, ic_ref, is_ref,
           rec_scr, h_scr):
    rec_scr[...] = (_dot(mr_ref[0], ic_ref[...]) + _dot(mi_ref[0], is_ref[...]))
    w0 = wg_ref[0, 0, 0]
    tt = tt_ref[...]

    def body(j, _):
        off = pl.multiple_of(j * 128, 128)
        row = idx_in_ref[0, :, pl.ds(off, 128)]
        rowb = jnp.broadcast_to(row, (64, 128))
        q = rowb >> 7
        r7 = rowb & 127
        res = jnp.take_along_axis(tt[:, 0:128], r7, axis=1)
        for qi in range(1, 4):
            gq = jnp.take_along_axis(tt[:, 128 * qi:128 * (qi + 1)], r7, axis=1)
            res = jnp.where(q == qi, gq, res)
        hv = w0 * res + rec_scr[:, pl.ds(off, 128)]
        hv = 0.5 * hv * (1.0 + jax.lax.erf(hv * np.float32(1.0 / np.sqrt(2.0))))
        h_scr[:, pl.ds(off, 128)] = hv
        return 0

    jax.lax.fori_loop(0, S // 128, body, 0)


def _first_body(x_ref, lwt_ref, lb_ref, fc_ref, fs_ref,
                fr_ref, fi_ref, idx_ref, hist_ref, hist_scr):
    x0 = _r(x_ref[0, 0:1, :])
    x1 = _r(x_ref[0, 1:2, :])
    x2 = _r(x_ref[0, 2:3, :])
    lwt = _r(lwt_ref[...])
    h = (lwt[:, 0:1] * x0 + lwt[:, 1:2] * x1 + lwt[:, 2:3] * x2
         + lb_ref[...])
    _analysis(h, fc_ref, fs_ref, fr_ref, fi_ref, idx_ref, hist_ref, hist_scr)


def _mid_body(fr_ref, fi_ref, hist_ref,
              emb512_ref, spw_ref, spwt_ref, sembt_ref, spbr_ref, spbc_ref,
              fremb_ref, frw_ref, frb_ref,
              gw1_ref, gb1_ref, gw2_ref, gb2_ref, wr_ref, wi_ref,
              mr_ref, mi_ref, wg_ref, tt_ref, ofr_scr, ofi_scr, mm_scr, fm_scr):
    t512 = _dot(emb512_ref[...], spw_ref[...], _DEF) + spbr_ref[...]
    tt_ref[...] = _dot(spwt_ref[...], sembt_ref[...], _DEF) + spbc_ref[...]
    hist = hist_ref[:, 0, :]
    sp_mean = _dot(hist, t512, _HI) * np.float32(1.0 / S)
    frv = fr_ref[...]
    fiv = fi_ref[...]
    mag = jnp.mean(jnp.sqrt(frv * frv + fiv * fiv), axis=1)
    idx_fr = jnp.sum((mag * 1000.0).astype(jnp.int32), axis=1,
                     keepdims=True) % NP_FR
    oh = (idx_fr == jax.lax.broadcasted_iota(jnp.int32, (1, NP_FR), 1))
    e = _dot(oh.astype(jnp.float32), fremb_ref[...], _HI)
    projf = _dot(e, frw_ref[...], _DEF) + frb_ref[...]
    for hh in range(HEADS):
        xr = _r(frv[:, CH * hh:CH * (hh + 1), :])
        xi = _r(fiv[:, CH * hh:CH * (hh + 1), :])
        for o in range(CH):
            w2r = _r(wr_ref[hh, :, o, :])
            w2i = _r(wi_ref[hh, :, o, :])
            vr = (jnp.sum(xr * w2r[None, :, :], axis=1)
                  - jnp.sum(xi * w2i[None, :, :], axis=1))
            vi = (jnp.sum(xr * w2i[None, :, :], axis=1)
                  + jnp.sum(xi * w2r[None, :, :], axis=1))
            c = CH * hh + o
            ofr_scr[:, c, :] = vr
            ofi_scr[:, c, :] = vi
            mm_scr[:, c:c + 1] = vr[:, 0:1] * np.float32(1.0 / S)
            fm_scr[:, c:c + 1] = projf[:, MODES * c:MODES * c + 1] * np.float32(1.0 / S)
    g = jnp.concatenate([sp_mean, mm_scr[...], fm_scr[...]], axis=1)
    h1 = jnp.maximum(_dot(g, gw1_ref[...], _DEF) + gb1_ref[...], 0.0)
    z = _dot(h1, gw2_ref[...], _DEF) + gb2_ref[...]
    zm = jnp.max(z, axis=1, keepdims=True)
    ez = jnp.exp(z - zm)
    w = ez / jnp.sum(ez, axis=1, keepdims=True)
    w0 = w[:, 0:1]
    w1 = w[:, 1:2]
    w2 = w[:, 2:3]
    wg_ref[...] = jnp.broadcast_to(w0[:, :, None], (B, 8, 128))
    mr_ref[...] = w1[:, :, None] * ofr_scr[...]
    mi_ref[...] = w1[:, :, None] * ofi_scr[...]
    for c in range(HID):
        mr_ref[:, c:c + 1, :] += (w2 * projf[:, MODES * c:MODES * (c + 1)])[:, None, :]


def _fused_body(idx_in_ref, mr_ref, mi_ref, wg_ref, tt_ref, ic_ref, is_ref,
                fc_ref, fs_ref, fr_ref, fi_ref, idx_ref, hist_ref,
                rec_scr, h_scr, hist_scr):
    _apply(idx_in_ref, mr_ref, mi_ref, wg_ref, tt_ref, ic_ref, is_ref,
           rec_scr, h_scr)
    _analysis(h_scr[...], fc_ref, fs_ref, fr_ref, fi_ref, idx_ref, hist_ref,
              hist_scr)


def _final_body(idx_in_ref, mr_ref, mi_ref, wg_ref, tt_ref, ic_ref, is_ref,
                pwt_ref, pb_ref, out_ref, rec_scr, h_scr):
    _apply(idx_in_ref, mr_ref, mi_ref, wg_ref, tt_ref, ic_ref, is_ref,
           rec_scr, h_scr)
    h = _r(h_scr[...])
    o = jnp.sum(h * _r(pwt_ref[...]), axis=0, keepdims=True)
    out_ref[0] = o + pb_ref[0, 0]


def _b_spec(shape):
    return pl.BlockSpec((1,) + shape, lambda b: (b,) + (0,) * len(shape))


def _w_spec(shape):
    return pl.BlockSpec(shape, lambda b: (0,) * len(shape))


_PAR = pltpu.CompilerParams(dimension_semantics=("parallel",))


def kernel(x, lift_W, lift_b, proj_W, proj_b, sp_emb, sp_W, sp_b,
           fr_emb, fr_W, fr_b, g_W1, g_b1, g_W2, g_b2, mhf_Wr, mhf_Wi):
    f32 = jnp.float32
    fc = jnp.asarray(_FC)
    fs = jnp.asarray(_FS)
    ic = jnp.asarray(_IC)
    is_ = jnp.asarray(_IS)
    lwt = lift_W.T
    lb = lift_b[:, None]
    pwt = proj_W
    pb = proj_b[:, None]

    fshape = jax.ShapeDtypeStruct((B, HID, MODES), f32)
    ishape = jax.ShapeDtypeStruct((B, 1, S), jnp.int32)
    hshape = jax.ShapeDtypeStruct((B, 1, NBINS), f32)

    fr, fi, idx, hist = pl.pallas_call(
        _first_body,
        grid=(B,),
        in_specs=[_b_spec((3, S)), _w_spec((HID, 3)), _w_spec((HID, 1)),
                  _w_spec((S, MODES)), _w_spec((S, MODES))],
        out_specs=[_b_spec((HID, MODES)), _b_spec((HID, MODES)),
                   _b_spec((1, S)), _b_spec((1, NBINS))],
        out_shape=[fshape, fshape, ishape, hshape],
        scratch_shapes=[pltpu.VMEM((NBINS, 128), f32)],
        compiler_params=_PAR,
    )(x, lwt, lb, fc, fs)

    for i in range(L):
        emb512 = jnp.pad(sp_emb[i], ((0, NBINS - NP_SP), (0, 0)))
        args = (fr, fi, hist, emb512, sp_W[i], sp_W[i].T, emb512.T,
                sp_b[i][None, :], sp_b[i][:, None],
                fr_emb[i], fr_W[i], fr_b[i][None, :],
                g_W1[i], g_b1[i][None, :], g_W2[i], g_b2[i][None, :],
                mhf_Wr[i], mhf_Wi[i])
        mr, mi, wg, tt = pl.pallas_call(
            _mid_body,
            out_shape=[jax.ShapeDtypeStruct((B, HID, MODES), f32),
                       jax.ShapeDtypeStruct((B, HID, MODES), f32),
                       jax.ShapeDtypeStruct((B, 8, 128), f32),
                       jax.ShapeDtypeStruct((HID, NBINS), f32)],
            scratch_shapes=[pltpu.VMEM((B, HID, MODES), f32),
                            pltpu.VMEM((B, HID, MODES), f32),
                            pltpu.VMEM((B, HID), f32),
                            pltpu.VMEM((B, HID), f32)],
        )(*args)
        if i < L - 1:
            fr, fi, idx, hist = pl.pallas_call(
                _fused_body,
                grid=(B,),
                in_specs=[_b_spec((1, S)), _b_spec((HID, MODES)),
                          _b_spec((HID, MODES)), _b_spec((8, 128)),
                          _w_spec((HID, NBINS)),
                          _w_spec((MODES, S)), _w_spec((MODES, S)),
                          _w_spec((S, MODES)), _w_spec((S, MODES))],
                out_specs=[_b_spec((HID, MODES)), _b_spec((HID, MODES)),
                           _b_spec((1, S)), _b_spec((1, NBINS))],
                out_shape=[fshape, fshape, ishape, hshape],
                scratch_shapes=[pltpu.VMEM((HID, S), f32),
                                pltpu.VMEM((HID, S), f32),
                                pltpu.VMEM((NBINS, 128), f32)],
                compiler_params=_PAR,
            )(idx, mr, mi, wg, tt, ic, is_, fc, fs)
        else:
            out = pl.pallas_call(
                _final_body,
                grid=(B,),
                in_specs=[_b_spec((1, S)), _b_spec((HID, MODES)),
                          _b_spec((HID, MODES)), _b_spec((8, 128)),
                          _w_spec((HID, NBINS)),
                          _w_spec((MODES, S)), _w_spec((MODES, S)),
                          _w_spec((HID, 1)), _w_spec((1, 1))],
                out_specs=[_b_spec((1, S))],
                out_shape=[jax.ShapeDtypeStruct((B, 1, S), f32)],
                scratch_shapes=[pltpu.VMEM((HID, S), f32),
                                pltpu.VMEM((HID, S), f32)],
                compiler_params=_PAR,
            )(idx, mr, mi, wg, tt, ic, is_, pwt, pb)[0]
    return out


# speedup vs baseline: 12.9505x; 1.4758x over previous
import numpy as np
import jax
import jax.numpy as jnp
from jax.experimental import pallas as pl
from jax.experimental.pallas import tpu as pltpu

L = 4
HID = 64
MODES = 16
HEADS = 4
CH = HID // HEADS
NP_SP = 500
NP_FR = 200
ED = 8
PRIME = 31.0
S = 16384
B = 16
NBINS = 512

_HI = jax.lax.Precision.HIGHEST
_H3 = jax.lax.Precision.HIGH

_s = np.arange(S, dtype=np.int64)
_k = np.arange(MODES, dtype=np.int64)
_ang = 2.0 * np.pi * ((_s[:, None] * _k[None, :]) % S).astype(np.float64) / S
_FC = np.cos(_ang).astype(np.float32)
_FS = (-np.sin(_ang)).astype(np.float32)
_wk = np.full((MODES, 1), 2.0 / S)
_wk[0, 0] = 1.0 / S
_IC = (np.cos(_ang).T * _wk).astype(np.float32)
_IS = (-np.sin(_ang).T * _wk).astype(np.float32)


def _dot(a, b, prec=_HI):
    return jax.lax.dot(a, b, precision=prec, preferred_element_type=jnp.float32)


_DEF = jax.lax.Precision.DEFAULT


def _r(x):
    return x.astype(jnp.bfloat16).astype(jnp.float32)


def _analysis(h, fc_ref, fs_ref, fr_ref, fi_ref, idx_ref, hist_ref, hist_scr):
    fr_ref[0] = _dot(h, fc_ref[...], _DEF)
    fi_ref[0] = _dot(h, fs_ref[...], _DEF)
    cs = jnp.sum(h, axis=0, keepdims=True)
    c0 = cs[0, 0]
    cL = cs[0, S - 1]
    iot = jax.lax.broadcasted_iota(jnp.int32, (1, S), 1)
    r2 = jnp.roll(cs, 2, axis=1)
    r1 = jnp.roll(cs, 1, axis=1)
    rm1 = jnp.roll(cs, -1, axis=1)
    a = jnp.where(iot >= 2, r2, c0)
    bt = jnp.where(iot >= 1, r1, c0)
    d = jnp.where(iot <= S - 2, rm1, cL)
    wsum = a + bt + cs + d
    idx = (wsum * PRIME).astype(jnp.int32) % NP_SP
    idx = jnp.clip(idx, 0, NP_SP - 1)
    idx_ref[0] = idx
    hist_scr[...] = jnp.zeros((NBINS, 128), jnp.float32)
    bins = jax.lax.broadcasted_iota(jnp.int32, (NBINS, 1), 0)

    def hbody(j, _):
        ch = idx_ref[0, :, pl.ds(j * 128, 128)]
        hist_scr[...] += (bins == ch).astype(jnp.float32)
        return 0

    jax.lax.fori_loop(0, S // 128, hbody, 0)
    ht = jnp.transpose(hist_scr[...])
    hist_ref[0] = jnp.sum(ht, axis=0, keepdims=True)


def _apply(idx_in_ref, mr_ref, mi_ref, wg_ref, tt_ref, ic_ref, is_ref,
           rec_scr, h_scr):
    rec_scr[...] = (_dot(mr_ref[0], ic_ref[...], _DEF)
                    + _dot(mi_ref[0], is_ref[...], _DEF))
    w0 = wg_ref[0, 0, 0]
    tt = tt_ref[...]

    def body(j, _):
        off = pl.multiple_of(j * 128, 128)
        row = idx_in_ref[0, :, pl.ds(off, 128)]
        rowb = jnp.broadcast_to(row, (64, 128))
        q = rowb >> 7
        r7 = rowb & 127
        res = jnp.take_along_axis(tt[:, 0:128], r7, axis=1)
        for qi in range(1, 4):
            gq = jnp.take_along_axis(tt[:, 128 * qi:128 * (qi + 1)], r7, axis=1)
            res = jnp.where(q == qi, gq, res)
        hv = w0 * res + rec_scr[:, pl.ds(off, 128)]
        hv = 0.5 * hv * (1.0 + jax.lax.erf(hv * np.float32(1.0 / np.sqrt(2.0))))
        h_scr[:, pl.ds(off, 128)] = hv
        return 0

    jax.lax.fori_loop(0, S // 128, body, 0)


def _first_body(x_ref, lwt_ref, lb_ref, fc_ref, fs_ref,
                fr_ref, fi_ref, idx_ref, hist_ref, hist_scr):
    x0 = _r(x_ref[0, 0:1, :])
    x1 = _r(x_ref[0, 1:2, :])
    x2 = _r(x_ref[0, 2:3, :])
    lwt = _r(lwt_ref[...])
    h = (lwt[:, 0:1] * x0 + lwt[:, 1:2] * x1 + lwt[:, 2:3] * x2
         + lb_ref[...])
    _analysis(h, fc_ref, fs_ref, fr_ref, fi_ref, idx_ref, hist_ref, hist_scr)


def _mid_body(fr_ref, fi_ref, hist_ref,
              emb512_ref, spw_ref, spwt_ref, sembt_ref, spbr_ref, spbc_ref,
              fremb_ref, frw_ref, frb_ref,
              gw1_ref, gb1_ref, gw2_ref, gb2_ref, wr_ref, wi_ref,
              mr_ref, mi_ref, wg_ref, tt_ref, ofr_scr, ofi_scr, mm_scr, fm_scr):
    t512 = _dot(emb512_ref[...], spw_ref[...], _DEF) + spbr_ref[...]
    tt_ref[...] = _dot(spwt_ref[...], sembt_ref[...], _DEF) + spbc_ref[...]
    hist = hist_ref[:, 0, :]
    sp_mean = _dot(hist, t512, _HI) * np.float32(1.0 / S)
    frv = fr_ref[...]
    fiv = fi_ref[...]
    mag = jnp.mean(jnp.sqrt(frv * frv + fiv * fiv), axis=1)
    idx_fr = jnp.sum((mag * 1000.0).astype(jnp.int32), axis=1,
                     keepdims=True) % NP_FR
    oh = (idx_fr == jax.lax.broadcasted_iota(jnp.int32, (1, NP_FR), 1))
    e = _dot(oh.astype(jnp.float32), fremb_ref[...], _HI)
    projf = _dot(e, frw_ref[...], _DEF) + frb_ref[...]
    for hh in range(HEADS):
        xr = _r(frv[:, CH * hh:CH * (hh + 1), :])
        xi = _r(fiv[:, CH * hh:CH * (hh + 1), :])
        for o in range(CH):
            w2r = _r(wr_ref[hh, :, o, :])
            w2i = _r(wi_ref[hh, :, o, :])
            vr = (jnp.sum(xr * w2r[None, :, :], axis=1)
                  - jnp.sum(xi * w2i[None, :, :], axis=1))
            vi = (jnp.sum(xr * w2i[None, :, :], axis=1)
                  + jnp.sum(xi * w2r[None, :, :], axis=1))
            c = CH * hh + o
            ofr_scr[:, c, :] = vr
            ofi_scr[:, c, :] = vi
            mm_scr[:, c:c + 1] = vr[:, 0:1] * np.float32(1.0 / S)
            fm_scr[:, c:c + 1] = projf[:, MODES * c:MODES * c + 1] * np.float32(1.0 / S)
    g = jnp.concatenate([sp_mean, mm_scr[...], fm_scr[...]], axis=1)
    h1 = jnp.maximum(_dot(g, gw1_ref[...], _DEF) + gb1_ref[...], 0.0)
    z = _dot(h1, gw2_ref[...], _DEF) + gb2_ref[...]
    zm = jnp.max(z, axis=1, keepdims=True)
    ez = jnp.exp(z - zm)
    w = ez / jnp.sum(ez, axis=1, keepdims=True)
    w0 = w[:, 0:1]
    w1 = w[:, 1:2]
    w2 = w[:, 2:3]
    wg_ref[...] = jnp.broadcast_to(w0[:, :, None], (B, 8, 128))
    mr_ref[...] = w1[:, :, None] * ofr_scr[...]
    mi_ref[...] = w1[:, :, None] * ofi_scr[...]
    for c in range(HID):
        mr_ref[:, c:c + 1, :] += (w2 * projf[:, MODES * c:MODES * (c + 1)])[:, None, :]


def _fused_body(idx_in_ref, mr_ref, mi_ref, wg_ref, tt_ref, ic_ref, is_ref,
                fc_ref, fs_ref, fr_ref, fi_ref, idx_ref, hist_ref,
                rec_scr, h_scr, hist_scr):
    _apply(idx_in_ref, mr_ref, mi_ref, wg_ref, tt_ref, ic_ref, is_ref,
           rec_scr, h_scr)
    _analysis(h_scr[...], fc_ref, fs_ref, fr_ref, fi_ref, idx_ref, hist_ref,
              hist_scr)


def _final_body(idx_in_ref, mr_ref, mi_ref, wg_ref, tt_ref, ic_ref, is_ref,
                pwt_ref, pb_ref, out_ref, rec_scr, h_scr):
    _apply(idx_in_ref, mr_ref, mi_ref, wg_ref, tt_ref, ic_ref, is_ref,
           rec_scr, h_scr)
    h = _r(h_scr[...])
    o = jnp.sum(h * _r(pwt_ref[...]), axis=0, keepdims=True)
    out_ref[0] = o + pb_ref[0, 0]


def _b_spec(shape):
    return pl.BlockSpec((1,) + shape, lambda b: (b,) + (0,) * len(shape))


def _w_spec(shape):
    return pl.BlockSpec(shape, lambda b: (0,) * len(shape))


_PAR = pltpu.CompilerParams(dimension_semantics=("parallel",))


def kernel(x, lift_W, lift_b, proj_W, proj_b, sp_emb, sp_W, sp_b,
           fr_emb, fr_W, fr_b, g_W1, g_b1, g_W2, g_b2, mhf_Wr, mhf_Wi):
    f32 = jnp.float32
    fc = jnp.asarray(_FC)
    fs = jnp.asarray(_FS)
    ic = jnp.asarray(_IC)
    is_ = jnp.asarray(_IS)
    lwt = lift_W.T
    lb = lift_b[:, None]
    pwt = proj_W
    pb = proj_b[:, None]

    fshape = jax.ShapeDtypeStruct((B, HID, MODES), f32)
    ishape = jax.ShapeDtypeStruct((B, 1, S), jnp.int32)
    hshape = jax.ShapeDtypeStruct((B, 1, NBINS), f32)

    fr, fi, idx, hist = pl.pallas_call(
        _first_body,
        grid=(B,),
        in_specs=[_b_spec((3, S)), _w_spec((HID, 3)), _w_spec((HID, 1)),
                  _w_spec((S, MODES)), _w_spec((S, MODES))],
        out_specs=[_b_spec((HID, MODES)), _b_spec((HID, MODES)),
                   _b_spec((1, S)), _b_spec((1, NBINS))],
        out_shape=[fshape, fshape, ishape, hshape],
        scratch_shapes=[pltpu.VMEM((NBINS, 128), f32)],
        compiler_params=_PAR,
    )(x, lwt, lb, fc, fs)

    for i in range(L):
        emb512 = jnp.pad(sp_emb[i], ((0, NBINS - NP_SP), (0, 0)))
        args = (fr, fi, hist, emb512, sp_W[i], sp_W[i].T, emb512.T,
                sp_b[i][None, :], sp_b[i][:, None],
                fr_emb[i], fr_W[i], fr_b[i][None, :],
                g_W1[i], g_b1[i][None, :], g_W2[i], g_b2[i][None, :],
                mhf_Wr[i], mhf_Wi[i])
        mr, mi, wg, tt = pl.pallas_call(
            _mid_body,
            out_shape=[jax.ShapeDtypeStruct((B, HID, MODES), f32),
                       jax.ShapeDtypeStruct((B, HID, MODES), f32),
                       jax.ShapeDtypeStruct((B, 8, 128), f32),
                       jax.ShapeDtypeStruct((HID, NBINS), f32)],
            scratch_shapes=[pltpu.VMEM((B, HID, MODES), f32),
                            pltpu.VMEM((B, HID, MODES), f32),
                            pltpu.VMEM((B, HID), f32),
                            pltpu.VMEM((B, HID), f32)],
        )(*args)
        if i < L - 1:
            fr, fi, idx, hist = pl.pallas_call(
                _fused_body,
                grid=(B,),
                in_specs=[_b_spec((1, S)), _b_spec((HID, MODES)),
                          _b_spec((HID, MODES)), _b_spec((8, 128)),
                          _w_spec((HID, NBINS)),
                          _w_spec((MODES, S)), _w_spec((MODES, S)),
                          _w_spec((S, MODES)), _w_spec((S, MODES))],
                out_specs=[_b_spec((HID, MODES)), _b_spec((HID, MODES)),
                           _b_spec((1, S)), _b_spec((1, NBINS))],
                out_shape=[fshape, fshape, ishape, hshape],
                scratch_shapes=[pltpu.VMEM((HID, S), f32),
                                pltpu.VMEM((HID, S), f32),
                                pltpu.VMEM((NBINS, 128), f32)],
                compiler_params=_PAR,
            )(idx, mr, mi, wg, tt, ic, is_, fc, fs)
        else:
            out = pl.pallas_call(
                _final_body,
                grid=(B,),
                in_specs=[_b_spec((1, S)), _b_spec((HID, MODES)),
                          _b_spec((HID, MODES)), _b_spec((8, 128)),
                          _w_spec((HID, NBINS)),
                          _w_spec((MODES, S)), _w_spec((MODES, S)),
                          _w_spec((HID, 1)), _w_spec((1, 1))],
                out_specs=[_b_spec((1, S))],
                out_shape=[jax.ShapeDtypeStruct((B, 1, S), f32)],
                scratch_shapes=[pltpu.VMEM((HID, S), f32),
                                pltpu.VMEM((HID, S), f32)],
                compiler_params=_PAR,
            )(idx, mr, mi, wg, tt, ic, is_, pwt, pb)[0]
    return out


# speedup vs baseline: 16.1925x; 1.2503x over previous
import numpy as np
import jax
import jax.numpy as jnp
from jax.experimental import pallas as pl
from jax.experimental.pallas import tpu as pltpu

L = 4
HID = 64
MODES = 16
HEADS = 4
CH = HID // HEADS
NP_SP = 500
NP_FR = 200
ED = 8
PRIME = 31.0
S = 16384
B = 16
NBINS = 512

_HI = jax.lax.Precision.HIGHEST
_H3 = jax.lax.Precision.HIGH

_s = np.arange(S, dtype=np.int64)
_k = np.arange(MODES, dtype=np.int64)
_ang = 2.0 * np.pi * ((_s[:, None] * _k[None, :]) % S).astype(np.float64) / S
_FC = np.cos(_ang).astype(np.float32)
_FS = (-np.sin(_ang)).astype(np.float32)
_wk = np.full((MODES, 1), 2.0 / S)
_wk[0, 0] = 1.0 / S
_IC = (np.cos(_ang).T * _wk).astype(np.float32)
_IS = (-np.sin(_ang).T * _wk).astype(np.float32)


def _dot(a, b, prec=_HI):
    return jax.lax.dot(a, b, precision=prec, preferred_element_type=jnp.float32)


_DEF = jax.lax.Precision.DEFAULT


def _r(x):
    return x.astype(jnp.bfloat16).astype(jnp.float32)


def _analysis(h, fc_ref, fs_ref, fr_ref, fi_ref, idx_ref, hist_ref):
    fr_ref[0] = _dot(h, fc_ref[...], _DEF)
    fi_ref[0] = _dot(h, fs_ref[...], _DEF)
    cs = jnp.sum(h, axis=0, keepdims=True)
    c0 = cs[0, 0]
    cL = cs[0, S - 1]
    iot = jax.lax.broadcasted_iota(jnp.int32, (1, S), 1)
    r2 = jnp.roll(cs, 2, axis=1)
    r1 = jnp.roll(cs, 1, axis=1)
    rm1 = jnp.roll(cs, -1, axis=1)
    a = jnp.where(iot >= 2, r2, c0)
    bt = jnp.where(iot >= 1, r1, c0)
    d = jnp.where(iot <= S - 2, rm1, cL)
    wsum = a + bt + cs + d
    idx = (wsum * PRIME).astype(jnp.int32) % NP_SP
    idx = jnp.clip(idx, 0, NP_SP - 1)
    idx_ref[0] = idx
    def _hq(qoff):
        bins = jax.lax.broadcasted_iota(jnp.int32, (128, 1), 0) + qoff

        def hbody(j, acc):
            ch = idx_ref[0, :, pl.ds(j * 128, 128)]
            return acc + (bins == ch).astype(jnp.float32)

        return jax.lax.fori_loop(0, S // 128, hbody,
                                 jnp.zeros((128, 128), jnp.float32))

    accq = jnp.concatenate([_hq(128 * q) for q in range(4)], axis=0)
    ht = jnp.transpose(accq)
    hist_ref[0] = jnp.sum(ht, axis=0, keepdims=True)


def _apply(idx_in_ref, mr_ref, mi_ref, wg_ref, tt_ref, ic_ref, is_ref,
           rec_scr, h_scr):
    rec_scr[...] = (_dot(mr_ref[0], ic_ref[...], _DEF)
                    + _dot(mi_ref[0], is_ref[...], _DEF))
    w0 = wg_ref[0, 0, 0]
    tt = tt_ref[...]

    def body(j, _):
        base = pl.multiple_of(j * 512, 512)
        for u in range(4):
            off = pl.multiple_of(base + 128 * u, 128)
            row = idx_in_ref[0, :, pl.ds(off, 128)]
            rowb = jnp.broadcast_to(row, (64, 128))
            q = rowb >> 7
            r7 = rowb & 127
            res = jnp.take_along_axis(tt[:, 0:128], r7, axis=1)
            for qi in range(1, 4):
                gq = jnp.take_along_axis(tt[:, 128 * qi:128 * (qi + 1)], r7,
                                         axis=1)
                res = jnp.where(q == qi, gq, res)
            hv = w0 * res + rec_scr[:, pl.ds(off, 128)]
            hv = 0.5 * hv * (1.0 + jax.lax.erf(hv * np.float32(1.0 / np.sqrt(2.0))))
            h_scr[:, pl.ds(off, 128)] = hv
        return 0

    jax.lax.fori_loop(0, S // 512, body, 0)


def _first_body(x_ref, lwt_ref, lb_ref, fc_ref, fs_ref,
                fr_ref, fi_ref, idx_ref, hist_ref):
    x0 = _r(x_ref[0, 0:1, :])
    x1 = _r(x_ref[0, 1:2, :])
    x2 = _r(x_ref[0, 2:3, :])
    lwt = _r(lwt_ref[...])
    h = (lwt[:, 0:1] * x0 + lwt[:, 1:2] * x1 + lwt[:, 2:3] * x2
         + lb_ref[...])
    _analysis(h, fc_ref, fs_ref, fr_ref, fi_ref, idx_ref, hist_ref)


def _mid_body(fr_ref, fi_ref, hist_ref,
              emb512_ref, spw_ref, spwt_ref, sembt_ref, spbr_ref, spbc_ref,
              fremb_ref, frw_ref, frb_ref,
              gw1_ref, gb1_ref, gw2_ref, gb2_ref, wr_ref, wi_ref,
              mr_ref, mi_ref, wg_ref, tt_ref, ofr_scr, ofi_scr, mm_scr, fm_scr):
    t512 = _dot(emb512_ref[...], spw_ref[...], _DEF) + spbr_ref[...]
    tt_ref[...] = _dot(spwt_ref[...], sembt_ref[...], _DEF) + spbc_ref[...]
    hist = hist_ref[:, 0, :]
    sp_mean = _dot(hist, t512, _HI) * np.float32(1.0 / S)
    frv = fr_ref[...]
    fiv = fi_ref[...]
    mag = jnp.mean(jnp.sqrt(frv * frv + fiv * fiv), axis=1)
    idx_fr = jnp.sum((mag * 1000.0).astype(jnp.int32), axis=1,
                     keepdims=True) % NP_FR
    oh = (idx_fr == jax.lax.broadcasted_iota(jnp.int32, (1, NP_FR), 1))
    e = _dot(oh.astype(jnp.float32), fremb_ref[...], _HI)
    projf = _dot(e, frw_ref[...], _DEF) + frb_ref[...]
    for hh in range(HEADS):
        xr = _r(frv[:, CH * hh:CH * (hh + 1), :])
        xi = _r(fiv[:, CH * hh:CH * (hh + 1), :])
        for o in range(CH):
            w2r = _r(wr_ref[hh, :, o, :])
            w2i = _r(wi_ref[hh, :, o, :])
            vr = (jnp.sum(xr * w2r[None, :, :], axis=1)
                  - jnp.sum(xi * w2i[None, :, :], axis=1))
            vi = (jnp.sum(xr * w2i[None, :, :], axis=1)
                  + jnp.sum(xi * w2r[None, :, :], axis=1))
            c = CH * hh + o
            ofr_scr[:, c, :] = vr
            ofi_scr[:, c, :] = vi
            mm_scr[:, c:c + 1] = vr[:, 0:1] * np.float32(1.0 / S)
            fm_scr[:, c:c + 1] = projf[:, MODES * c:MODES * c + 1] * np.float32(1.0 / S)
    g = jnp.concatenate([sp_mean, mm_scr[...], fm_scr[...]], axis=1)
    h1 = jnp.maximum(_dot(g, gw1_ref[...], _DEF) + gb1_ref[...], 0.0)
    z = _dot(h1, gw2_ref[...], _DEF) + gb2_ref[...]
    zm = jnp.max(z, axis=1, keepdims=True)
    ez = jnp.exp(z - zm)
    w = ez / jnp.sum(ez, axis=1, keepdims=True)
    w0 = w[:, 0:1]
    w1 = w[:, 1:2]
    w2 = w[:, 2:3]
    wg_ref[...] = jnp.broadcast_to(w0[:, :, None], (B, 8, 128))
    mr_ref[...] = w1[:, :, None] * ofr_scr[...]
    mi_ref[...] = w1[:, :, None] * ofi_scr[...]
    for c in range(HID):
        mr_ref[:, c:c + 1, :] += (w2 * projf[:, MODES * c:MODES * (c + 1)])[:, None, :]


def _fused_body(idx_in_ref, mr_ref, mi_ref, wg_ref, tt_ref, ic_ref, is_ref,
                fc_ref, fs_ref, fr_ref, fi_ref, idx_ref, hist_ref,
                rec_scr, h_scr):
    _apply(idx_in_ref, mr_ref, mi_ref, wg_ref, tt_ref, ic_ref, is_ref,
           rec_scr, h_scr)
    _analysis(h_scr[...], fc_ref, fs_ref, fr_ref, fi_ref, idx_ref, hist_ref)


def _final_body(idx_in_ref, mr_ref, mi_ref, wg_ref, tt_ref, ic_ref, is_ref,
                pwt_ref, pb_ref, out_ref, rec_scr, h_scr):
    _apply(idx_in_ref, mr_ref, mi_ref, wg_ref, tt_ref, ic_ref, is_ref,
           rec_scr, h_scr)
    h = _r(h_scr[...])
    o = jnp.sum(h * _r(pwt_ref[...]), axis=0, keepdims=True)
    out_ref[0] = o + pb_ref[0, 0]


def _b_spec(shape):
    return pl.BlockSpec((1,) + shape, lambda b: (b,) + (0,) * len(shape))


def _w_spec(shape):
    return pl.BlockSpec(shape, lambda b: (0,) * len(shape))


_PAR = pltpu.CompilerParams(dimension_semantics=("parallel",))


def kernel(x, lift_W, lift_b, proj_W, proj_b, sp_emb, sp_W, sp_b,
           fr_emb, fr_W, fr_b, g_W1, g_b1, g_W2, g_b2, mhf_Wr, mhf_Wi):
    f32 = jnp.float32
    fc = jnp.asarray(_FC)
    fs = jnp.asarray(_FS)
    ic = jnp.asarray(_IC)
    is_ = jnp.asarray(_IS)
    lwt = lift_W.T
    lb = lift_b[:, None]
    pwt = proj_W
    pb = proj_b[:, None]

    fshape = jax.ShapeDtypeStruct((B, HID, MODES), f32)
    ishape = jax.ShapeDtypeStruct((B, 1, S), jnp.int32)
    hshape = jax.ShapeDtypeStruct((B, 1, NBINS), f32)

    fr, fi, idx, hist = pl.pallas_call(
        _first_body,
        grid=(B,),
        in_specs=[_b_spec((3, S)), _w_spec((HID, 3)), _w_spec((HID, 1)),
                  _w_spec((S, MODES)), _w_spec((S, MODES))],
        out_specs=[_b_spec((HID, MODES)), _b_spec((HID, MODES)),
                   _b_spec((1, S)), _b_spec((1, NBINS))],
        out_shape=[fshape, fshape, ishape, hshape],
        compiler_params=_PAR,
    )(x, lwt, lb, fc, fs)

    for i in range(L):
        emb512 = jnp.pad(sp_emb[i], ((0, NBINS - NP_SP), (0, 0)))
        args = (fr, fi, hist, emb512, sp_W[i], sp_W[i].T, emb512.T,
                sp_b[i][None, :], sp_b[i][:, None],
                fr_emb[i], fr_W[i], fr_b[i][None, :],
                g_W1[i], g_b1[i][None, :], g_W2[i], g_b2[i][None, :],
                mhf_Wr[i], mhf_Wi[i])
        mr, mi, wg, tt = pl.pallas_call(
            _mid_body,
            out_shape=[jax.ShapeDtypeStruct((B, HID, MODES), f32),
                       jax.ShapeDtypeStruct((B, HID, MODES), f32),
                       jax.ShapeDtypeStruct((B, 8, 128), f32),
                       jax.ShapeDtypeStruct((HID, NBINS), f32)],
            scratch_shapes=[pltpu.VMEM((B, HID, MODES), f32),
                            pltpu.VMEM((B, HID, MODES), f32),
                            pltpu.VMEM((B, HID), f32),
                            pltpu.VMEM((B, HID), f32)],
        )(*args)
        if i < L - 1:
            fr, fi, idx, hist = pl.pallas_call(
                _fused_body,
                grid=(B,),
                in_specs=[_b_spec((1, S)), _b_spec((HID, MODES)),
                          _b_spec((HID, MODES)), _b_spec((8, 128)),
                          _w_spec((HID, NBINS)),
                          _w_spec((MODES, S)), _w_spec((MODES, S)),
                          _w_spec((S, MODES)), _w_spec((S, MODES))],
                out_specs=[_b_spec((HID, MODES)), _b_spec((HID, MODES)),
                           _b_spec((1, S)), _b_spec((1, NBINS))],
                out_shape=[fshape, fshape, ishape, hshape],
                scratch_shapes=[pltpu.VMEM((HID, S), f32),
                                pltpu.VMEM((HID, S), f32)],
                compiler_params=_PAR,
            )(idx, mr, mi, wg, tt, ic, is_, fc, fs)
        else:
            out = pl.pallas_call(
                _final_body,
                grid=(B,),
                in_specs=[_b_spec((1, S)), _b_spec((HID, MODES)),
                          _b_spec((HID, MODES)), _b_spec((8, 128)),
                          _w_spec((HID, NBINS)),
                          _w_spec((MODES, S)), _w_spec((MODES, S)),
                          _w_spec((HID, 1)), _w_spec((1, 1))],
                out_specs=[_b_spec((1, S))],
                out_shape=[jax.ShapeDtypeStruct((B, 1, S), f32)],
                scratch_shapes=[pltpu.VMEM((HID, S), f32),
                                pltpu.VMEM((HID, S), f32)],
                compiler_params=_PAR,
            )(idx, mr, mi, wg, tt, ic, is_, pwt, pb)[0]
    return out
